# Initial kernel scaffold; baseline (speedup 1.0000x reference)
import jax
import jax.numpy as jnp
from jax.experimental import pallas as pl


def kernel(x, edge_index, W, b, W1, b1, W2, b2):
    raise NotImplementedError("write your pallas kernel here")


# speedup vs baseline: 59.9833x; 59.9833x over previous
import functools

import jax
import jax.numpy as jnp
from jax.experimental import pallas as pl

N = 2048
F = 16
BM = 128
BN = 128


def _rne_bf16(v):
    u = jax.lax.bitcast_convert_type(v, jnp.uint32)
    u = u + jnp.uint32(0x7FFF) + ((u >> 16) & jnp.uint32(1))
    u = u & jnp.uint32(0xFFFF0000)
    return jax.lax.bitcast_convert_type(u, jnp.float32)


def _tile_score(x_rows, yt_cols, w2):
    acc = jnp.zeros((BM, BN), jnp.float32)
    w2r = _rne_bf16(w2)
    for k in range(F):
        t = x_rows[:, k : k + 1] + yt_cols[k : k + 1, :]
        acc = acc + w2r[0, k] * _rne_bf16(jnp.maximum(t, 0.0))
    return acc


def _pairs_body(a_i, bt_j, b_i, at_j, w2, b2, out_ref):
    ib = pl.program_id(0)
    jb = pl.program_id(1)

    def upper():
        return _tile_score(a_i[...], bt_j[...], w2[...])

    def lower():
        return _tile_score(b_i[...], at_j[...], w2[...])

    def diag():
        rows = jax.lax.broadcasted_iota(jnp.int32, (BM, BN), 0)
        cols = jax.lax.broadcasted_iota(jnp.int32, (BM, BN), 1)
        return jnp.where(rows < cols, upper(), lower())

    acc = jax.lax.cond(
        ib < jb, upper, lambda: jax.lax.cond(ib > jb, lower, diag)
    )
    score = acc + b2[0, 0]
    rows = ib * BM + jax.lax.broadcasted_iota(jnp.int32, (BM, BN), 0)
    cols = jb * BN + jax.lax.broadcasted_iota(jnp.int32, (BM, BN), 1)
    hit = jnp.logical_and(score > 0.0, rows != cols)
    out_ref[...] = hit.astype(jnp.float32)


def _all_pairs(A, B, W2, b2):
    w2 = W2.reshape(1, F)
    b2 = b2.reshape(1, 1)
    grid = (N // BM, N // BN)
    return pl.pallas_call(
        _pairs_body,
        grid=grid,
        in_specs=[
            pl.BlockSpec((BM, F), lambda i, j: (i, 0)),
            pl.BlockSpec((F, BN), lambda i, j: (0, j)),
            pl.BlockSpec((BM, F), lambda i, j: (i, 0)),
            pl.BlockSpec((F, BN), lambda i, j: (0, j)),
            pl.BlockSpec((1, F), lambda i, j: (0, 0)),
            pl.BlockSpec((1, 1), lambda i, j: (0, 0)),
        ],
        out_specs=pl.BlockSpec((BM, BN), lambda i, j: (i, j)),
        out_shape=jax.ShapeDtypeStruct((N, N), jnp.float32),
    )(A, B.T, B, A.T, w2, b2)


def kernel(x, edge_index, W, b, W1, b1, W2, b2):
    src = edge_index[0]
    dst = edge_index[1]
    hw = x @ W
    deg = jnp.zeros((N,), jnp.float32).at[dst].add(1.0) + 1.0
    dinv = jax.lax.rsqrt(deg)
    g = dinv[:, None] * hw
    acc = jnp.zeros((N, F), jnp.float32).at[dst].add(g[src])
    h = jnp.maximum(dinv[:, None] * (acc + g) + b[None, :], 0.0)
    A = h @ W1[:F] + b1[None, :]
    B = h @ W1[F:]
    return _all_pairs(A, B, W2, b2)


# speedup vs baseline: 150.2820x; 2.5054x over previous
import functools

import jax
import jax.numpy as jnp
from jax import lax
from jax.experimental import pallas as pl
from jax.experimental.pallas import tpu as pltpu
from jax.experimental.pallas import tpu_sc as plsc

N = 2048
E = 65536
D = 128
F = 16
BM = 128
BN = 128

NC = 2
NS = 16
NW = NC * NS
EPT = E // NW
NCH = EPT // 128
RPT = N // NS

_mesh = plsc.VectorSubcoreMesh(core_axis_name="c", subcore_axis_name="s")


def _rne_bf16(v):
    u = lax.bitcast_convert_type(v, jnp.uint32)
    u = u + jnp.uint32(0x7FFF) + ((u >> 16) & jnp.uint32(1))
    u = u & jnp.uint32(0xFFFF0000)
    return lax.bitcast_convert_type(u, jnp.float32)


def _dot(a, b):
    return lax.dot_general(
        _rne_bf16(a), _rne_bf16(b), (((1,), (0,)), ((), ())),
        precision=lax.Precision.HIGHEST,
    )


@functools.partial(
    pl.kernel,
    out_type=jax.ShapeDtypeStruct((NC, N), jnp.float32),
    mesh=_mesh,
    scratch_types=[
        pltpu.VMEM((NCH, 128), jnp.int32),
        pltpu.VMEM((128,), jnp.float32),
        pltpu.VMEM_SHARED((N,), jnp.float32),
    ],
)
def _sc_hist(dst_hbm, deg_out, idx_v, ones_v, deg_sh):
    c = lax.axis_index("c")
    s = lax.axis_index("s")
    w = s * NC + c
    base = w * EPT
    for i in range(8):
        ones_v[pl.ds(i * 16, 16)] = jnp.zeros((16,), jnp.float32)
    pltpu.sync_copy(ones_v, deg_sh.at[pl.ds(s * RPT, RPT)])
    for i in range(8):
        ones_v[pl.ds(i * 16, 16)] = jnp.ones((16,), jnp.float32)
    for j in range(NCH):
        pltpu.sync_copy(dst_hbm.at[pl.ds(base + j * 128, 128)], idx_v.at[j])
    plsc.subcore_barrier()
    for j in range(NCH):
        pltpu.sync_copy(ones_v, deg_sh.at[idx_v.at[j]], add=True)
    plsc.subcore_barrier()
    pltpu.sync_copy(
        deg_sh.at[pl.ds(s * RPT, RPT)], deg_out.at[c, pl.ds(s * RPT, RPT)]
    )


def _prep_body(deg_ref, x_ref, w_ref, g_ref, dinv_ref):
    deg = deg_ref[0:1, :] + deg_ref[1:2, :] + 1.0
    dinv = jnp.reshape(lax.rsqrt(deg), (N, 1))
    g_ref[...] = dinv * _dot(x_ref[...], w_ref[...])
    dinv_ref[...] = dinv


def _prep(deg_p, x, W):
    return pl.pallas_call(
        _prep_body,
        out_shape=(
            jax.ShapeDtypeStruct((N, F), jnp.float32),
            jax.ShapeDtypeStruct((N, 1), jnp.float32),
        ),
    )(deg_p, x, W)


@functools.partial(
    pl.kernel,
    out_type=jax.ShapeDtypeStruct((NC, N, F), jnp.float32),
    mesh=_mesh,
    scratch_types=[
        pltpu.VMEM((NCH, 128), jnp.int32),
        pltpu.VMEM((NCH, 128), jnp.int32),
        pltpu.VMEM((EPT, F), jnp.float32),
        pltpu.VMEM((RPT, F), jnp.float32),
        pltpu.VMEM_SHARED((N, F), jnp.float32),
        pltpu.SemaphoreType.DMA,
    ],
    compiler_params=pltpu.CompilerParams(use_tc_tiling_on_sc=False),
)
def _sc_scatter(src_hbm, dst_hbm, g_hbm, acc_out, si_v, di_v, rows_v, buf_v,
                acc_sh, sem):
    c = lax.axis_index("c")
    s = lax.axis_index("s")
    w = s * NC + c
    base = w * EPT
    for i in range(RPT):
        buf_v[i, :] = jnp.zeros((16,), jnp.float32)
    pltpu.sync_copy(buf_v, acc_sh.at[pl.ds(s * RPT, RPT)])
    for j in range(NCH):
        pltpu.sync_copy(src_hbm.at[pl.ds(base + j * 128, 128)], si_v.at[j])
        pltpu.sync_copy(dst_hbm.at[pl.ds(base + j * 128, 128)], di_v.at[j])
    waits = []
    for j in range(NCH):
        waits.append(
            pltpu.async_copy(
                g_hbm.at[si_v.at[j]], rows_v.at[pl.ds(j * 128, 128)], sem
            )
        )
    for wdma in waits:
        wdma.wait()
    plsc.subcore_barrier()
    for j in range(NCH):
        pltpu.sync_copy(
            rows_v.at[pl.ds(j * 128, 128)], acc_sh.at[di_v.at[j]], add=True
        )
    plsc.subcore_barrier()
    pltpu.sync_copy(
        acc_sh.at[pl.ds(s * RPT, RPT)], acc_out.at[c, pl.ds(s * RPT, RPT)]
    )


def _ab_body(acc_ref, g_ref, dinv_ref, b_ref, w1a_ref, w1b_ref, b1_ref,
             a_ref, bb_ref):
    pre = dinv_ref[...] * (acc_ref[0] + acc_ref[1] + g_ref[...]) + b_ref[...]
    h = jnp.maximum(pre, 0.0)
    a_ref[...] = _dot(h, w1a_ref[...]) + b1_ref[...]
    bb_ref[...] = _dot(h, w1b_ref[...])


def _ab(acc_p, g, dinv, b, W1, b1):
    return pl.pallas_call(
        _ab_body,
        out_shape=(
            jax.ShapeDtypeStruct((N, F), jnp.float32),
            jax.ShapeDtypeStruct((N, F), jnp.float32),
        ),
    )(acc_p, g, dinv, b.reshape(1, F), W1[:F], W1[F:], b1.reshape(1, F))


def _tile_score(x_rows, yt_cols, w2r):
    acc = jnp.zeros((BM, BN), jnp.float32)
    for k in range(F):
        t = x_rows[:, k : k + 1] + yt_cols[k : k + 1, :]
        acc = acc + w2r[0, k] * _rne_bf16(jnp.maximum(t, 0.0))
    return acc


def _pairs_body(a_i, bt_j, b_i, at_j, w2, b2, out_ref):
    ib = pl.program_id(0)
    jb = pl.program_id(1)
    w2r = _rne_bf16(w2[...])

    def upper():
        return _tile_score(a_i[...], bt_j[...], w2r)

    def lower():
        return _tile_score(b_i[...], at_j[...], w2r)

    def diag():
        rows = lax.broadcasted_iota(jnp.int32, (BM, BN), 0)
        cols = lax.broadcasted_iota(jnp.int32, (BM, BN), 1)
        return jnp.where(rows < cols, upper(), lower())

    acc = lax.cond(ib < jb, upper, lambda: lax.cond(ib > jb, lower, diag))
    score = acc + b2[0, 0]
    rows = ib * BM + lax.broadcasted_iota(jnp.int32, (BM, BN), 0)
    cols = jb * BN + lax.broadcasted_iota(jnp.int32, (BM, BN), 1)
    hit = jnp.logical_and(score > 0.0, rows != cols)
    out_ref[...] = hit.astype(jnp.float32)


def _all_pairs(A, B, W2, b2):
    w2 = W2.reshape(1, F)
    b2 = b2.reshape(1, 1)
    grid = (N // BM, N // BN)
    return pl.pallas_call(
        _pairs_body,
        grid=grid,
        in_specs=[
            pl.BlockSpec((BM, F), lambda i, j: (i, 0)),
            pl.BlockSpec((F, BN), lambda i, j: (0, j)),
            pl.BlockSpec((BM, F), lambda i, j: (i, 0)),
            pl.BlockSpec((F, BN), lambda i, j: (0, j)),
            pl.BlockSpec((1, F), lambda i, j: (0, 0)),
            pl.BlockSpec((1, 1), lambda i, j: (0, 0)),
        ],
        out_specs=pl.BlockSpec((BM, BN), lambda i, j: (i, j)),
        out_shape=jax.ShapeDtypeStruct((N, N), jnp.float32),
    )(A, B.T, B, A.T, w2, b2)


def kernel(x, edge_index, W, b, W1, b1, W2, b2):
    src = edge_index[0]
    dst = edge_index[1]
    deg_p = _sc_hist(dst)
    g, dinv = _prep(deg_p, x, W)
    acc_p = _sc_scatter(src, dst, g)
    A, B = _ab(acc_p, g, dinv, b, W1, b1)
    return _all_pairs(A, B, W2, b2)


# speedup vs baseline: 152.1548x; 1.0125x over previous
import functools

import jax
import jax.numpy as jnp
from jax import lax
from jax.experimental import pallas as pl
from jax.experimental.pallas import tpu as pltpu
from jax.experimental.pallas import tpu_sc as plsc

N = 2048
E = 65536
D = 128
F = 16
BM = 128
BN = 128

NC = 2
NS = 16
NW = NC * NS
EPT = E // NW
NCH = EPT // 128
RPT = N // NS

_mesh = plsc.VectorSubcoreMesh(core_axis_name="c", subcore_axis_name="s")


def _rne_bf16(v):
    u = lax.bitcast_convert_type(v, jnp.uint32)
    u = u + jnp.uint32(0x7FFF) + ((u >> 16) & jnp.uint32(1))
    u = u & jnp.uint32(0xFFFF0000)
    return lax.bitcast_convert_type(u, jnp.float32)


def _dot(a, b):
    return lax.dot_general(
        _rne_bf16(a), _rne_bf16(b), (((1,), (0,)), ((), ())),
        precision=lax.Precision.HIGHEST,
    )


@functools.partial(
    pl.kernel,
    out_type=jax.ShapeDtypeStruct((NC, N), jnp.float32),
    mesh=_mesh,
    scratch_types=[
        pltpu.VMEM((NCH, 128), jnp.int32),
        pltpu.VMEM((128,), jnp.float32),
        pltpu.VMEM_SHARED((N,), jnp.float32),
    ],
)
def _sc_hist(dst_hbm, deg_out, idx_v, ones_v, deg_sh):
    c = lax.axis_index("c")
    s = lax.axis_index("s")
    w = s * NC + c
    base = w * EPT
    for i in range(8):
        ones_v[pl.ds(i * 16, 16)] = jnp.zeros((16,), jnp.float32)
    pltpu.sync_copy(ones_v, deg_sh.at[pl.ds(s * RPT, RPT)])
    for i in range(8):
        ones_v[pl.ds(i * 16, 16)] = jnp.ones((16,), jnp.float32)
    for j in range(NCH):
        pltpu.sync_copy(dst_hbm.at[pl.ds(base + j * 128, 128)], idx_v.at[j])
    plsc.subcore_barrier()
    for j in range(NCH):
        pltpu.sync_copy(ones_v, deg_sh.at[idx_v.at[j]], add=True)
    plsc.subcore_barrier()
    pltpu.sync_copy(
        deg_sh.at[pl.ds(s * RPT, RPT)], deg_out.at[c, pl.ds(s * RPT, RPT)]
    )


def _prep_body(deg_ref, x_ref, w_ref, g_ref, dinv_ref):
    deg = deg_ref[0:1, :] + deg_ref[1:2, :] + 1.0
    dinv = jnp.reshape(lax.rsqrt(deg), (N, 1))
    g_ref[...] = dinv * _dot(x_ref[...], w_ref[...])
    dinv_ref[...] = dinv


def _prep(deg_p, x, W):
    return pl.pallas_call(
        _prep_body,
        out_shape=(
            jax.ShapeDtypeStruct((N, F), jnp.float32),
            jax.ShapeDtypeStruct((N, 1), jnp.float32),
        ),
    )(deg_p, x, W)


@functools.partial(
    pl.kernel,
    out_type=jax.ShapeDtypeStruct((NC, N, F), jnp.float32),
    mesh=_mesh,
    scratch_types=[
        pltpu.VMEM((NCH, 128), jnp.int32),
        pltpu.VMEM((NCH, 128), jnp.int32),
        pltpu.VMEM((EPT, F), jnp.float32),
        pltpu.VMEM((RPT, F), jnp.float32),
        pltpu.VMEM_SHARED((N, F), jnp.float32),
        pltpu.SemaphoreType.DMA,
    ],
    compiler_params=pltpu.CompilerParams(use_tc_tiling_on_sc=False),
)
def _sc_scatter(src_hbm, dst_hbm, g_hbm, acc_out, si_v, di_v, rows_v, buf_v,
                acc_sh, sem):
    c = lax.axis_index("c")
    s = lax.axis_index("s")
    w = s * NC + c
    base = w * EPT
    for i in range(RPT):
        buf_v[i, :] = jnp.zeros((16,), jnp.float32)
    pltpu.sync_copy(buf_v, acc_sh.at[pl.ds(s * RPT, RPT)])
    for j in range(NCH):
        pltpu.sync_copy(src_hbm.at[pl.ds(base + j * 128, 128)], si_v.at[j])
        pltpu.sync_copy(dst_hbm.at[pl.ds(base + j * 128, 128)], di_v.at[j])
    waits = []
    for j in range(NCH):
        waits.append(
            pltpu.async_copy(
                g_hbm.at[si_v.at[j]], rows_v.at[pl.ds(j * 128, 128)], sem
            )
        )
    for wdma in waits:
        wdma.wait()
    plsc.subcore_barrier()
    for j in range(NCH):
        pltpu.sync_copy(
            rows_v.at[pl.ds(j * 128, 128)], acc_sh.at[di_v.at[j]], add=True
        )
    plsc.subcore_barrier()
    pltpu.sync_copy(
        acc_sh.at[pl.ds(s * RPT, RPT)], acc_out.at[c, pl.ds(s * RPT, RPT)]
    )


def _ab_body(acc_ref, g_ref, dinv_ref, b_ref, w1a_ref, w1b_ref, b1_ref,
             a_ref, bb_ref):
    pre = dinv_ref[...] * (acc_ref[0] + acc_ref[1] + g_ref[...]) + b_ref[...]
    h = jnp.maximum(pre, 0.0)
    a_ref[...] = _dot(h, w1a_ref[...]) + b1_ref[...]
    bb_ref[...] = _dot(h, w1b_ref[...])


def _ab(acc_p, g, dinv, b, W1, b1):
    return pl.pallas_call(
        _ab_body,
        out_shape=(
            jax.ShapeDtypeStruct((N, F), jnp.float32),
            jax.ShapeDtypeStruct((N, F), jnp.float32),
        ),
    )(acc_p, g, dinv, b.reshape(1, F), W1[:F], W1[F:], b1.reshape(1, F))


def _tile_score(x_rows, yt_cols, w2r):
    acc = jnp.zeros((BM, BN), jnp.float32)
    for k in range(F):
        t = x_rows[:, k : k + 1] + yt_cols[k : k + 1, :]
        tr = jnp.maximum(t, 0.0).astype(jnp.bfloat16).astype(jnp.float32)
        acc = acc + w2r[0, k] * tr
    return acc


def _pairs_body(a_i, bt_j, b_i, at_j, w2, b2, out_ref):
    ib = pl.program_id(0)
    jb = pl.program_id(1)
    w2r = w2[...]

    def upper():
        return _tile_score(a_i[...], bt_j[...], w2r)

    def lower():
        return _tile_score(b_i[...], at_j[...], w2r)

    def diag():
        rows = lax.broadcasted_iota(jnp.int32, (BM, BN), 0)
        cols = lax.broadcasted_iota(jnp.int32, (BM, BN), 1)
        return jnp.where(rows < cols, upper(), lower())

    acc = lax.cond(ib < jb, upper, lambda: lax.cond(ib > jb, lower, diag))
    score = acc + b2[0, 0]
    rows = ib * BM + lax.broadcasted_iota(jnp.int32, (BM, BN), 0)
    cols = jb * BN + lax.broadcasted_iota(jnp.int32, (BM, BN), 1)
    hit = jnp.logical_and(score > 0.0, rows != cols)
    out_ref[...] = hit.astype(jnp.float32)


def _all_pairs(A, B, W2, b2):
    w2 = _rne_bf16(W2.reshape(1, F))
    b2 = b2.reshape(1, 1)
    grid = (N // BM, N // BN)
    return pl.pallas_call(
        _pairs_body,
        grid=grid,
        in_specs=[
            pl.BlockSpec((BM, F), lambda i, j: (i, 0)),
            pl.BlockSpec((F, BN), lambda i, j: (0, j)),
            pl.BlockSpec((BM, F), lambda i, j: (i, 0)),
            pl.BlockSpec((F, BN), lambda i, j: (0, j)),
            pl.BlockSpec((1, F), lambda i, j: (0, 0)),
            pl.BlockSpec((1, 1), lambda i, j: (0, 0)),
        ],
        out_specs=pl.BlockSpec((BM, BN), lambda i, j: (i, j)),
        out_shape=jax.ShapeDtypeStruct((N, N), jnp.float32),
    )(A, B.T, B, A.T, w2, b2)


def kernel(x, edge_index, W, b, W1, b1, W2, b2):
    src = edge_index[0]
    dst = edge_index[1]
    deg_p = _sc_hist(dst)
    g, dinv = _prep(deg_p, x, W)
    acc_p = _sc_scatter(src, dst, g)
    A, B = _ab(acc_p, g, dinv, b, W1, b1)
    return _all_pairs(A, B, W2, b2)


# speedup vs baseline: 152.2007x; 1.0003x over previous
import functools

import jax
import jax.numpy as jnp
from jax import lax
from jax.experimental import pallas as pl
from jax.experimental.pallas import tpu as pltpu
from jax.experimental.pallas import tpu_sc as plsc

N = 2048
E = 65536
D = 128
F = 16
BM = 128
BN = 128

NC = 2
NS = 16
NW = NC * NS
EPT = E // NW
NCH = EPT // 128
RPT = N // NS

@functools.cache
def _mesh():
    return plsc.VectorSubcoreMesh(core_axis_name="c", subcore_axis_name="s")


def _rne_bf16(v):
    u = lax.bitcast_convert_type(v, jnp.uint32)
    u = u + jnp.uint32(0x7FFF) + ((u >> 16) & jnp.uint32(1))
    u = u & jnp.uint32(0xFFFF0000)
    return lax.bitcast_convert_type(u, jnp.float32)


def _dot(a, b):
    return lax.dot_general(
        _rne_bf16(a), _rne_bf16(b), (((1,), (0,)), ((), ())),
        precision=lax.Precision.HIGHEST,
    )


def _sc_hist(dst):
    return pl.kernel(
        _sc_hist_body,
        out_type=jax.ShapeDtypeStruct((NC, N), jnp.float32),
        mesh=_mesh(),
        scratch_types=[
            pltpu.VMEM((NCH, 128), jnp.int32),
            pltpu.VMEM((128,), jnp.float32),
            pltpu.VMEM_SHARED((N,), jnp.float32),
        ],
    )(dst)


def _sc_hist_body(dst_hbm, deg_out, idx_v, ones_v, deg_sh):
    c = lax.axis_index("c")
    s = lax.axis_index("s")
    w = s * NC + c
    base = w * EPT
    for i in range(8):
        ones_v[pl.ds(i * 16, 16)] = jnp.zeros((16,), jnp.float32)
    pltpu.sync_copy(ones_v, deg_sh.at[pl.ds(s * RPT, RPT)])
    for i in range(8):
        ones_v[pl.ds(i * 16, 16)] = jnp.ones((16,), jnp.float32)
    for j in range(NCH):
        pltpu.sync_copy(dst_hbm.at[pl.ds(base + j * 128, 128)], idx_v.at[j])
    plsc.subcore_barrier()
    for j in range(NCH):
        pltpu.sync_copy(ones_v, deg_sh.at[idx_v.at[j]], add=True)
    plsc.subcore_barrier()
    pltpu.sync_copy(
        deg_sh.at[pl.ds(s * RPT, RPT)], deg_out.at[c, pl.ds(s * RPT, RPT)]
    )


def _prep_body(deg_ref, x_ref, w_ref, g_ref, dinv_ref):
    deg = deg_ref[0:1, :] + deg_ref[1:2, :] + 1.0
    dinv = jnp.reshape(lax.rsqrt(deg), (N, 1))
    g_ref[...] = dinv * _dot(x_ref[...], w_ref[...])
    dinv_ref[...] = dinv


def _prep(deg_p, x, W):
    return pl.pallas_call(
        _prep_body,
        out_shape=(
            jax.ShapeDtypeStruct((N, F), jnp.float32),
            jax.ShapeDtypeStruct((N, 1), jnp.float32),
        ),
    )(deg_p, x, W)


def _sc_scatter(src, dst, g):
    return pl.kernel(
        _sc_scatter_body,
        out_type=jax.ShapeDtypeStruct((NC, N, F), jnp.float32),
        mesh=_mesh(),
        scratch_types=[
            pltpu.VMEM((NCH, 128), jnp.int32),
            pltpu.VMEM((NCH, 128), jnp.int32),
            pltpu.VMEM((EPT, F), jnp.float32),
            pltpu.VMEM((RPT, F), jnp.float32),
            pltpu.VMEM_SHARED((N, F), jnp.float32),
            pltpu.SemaphoreType.DMA,
        ],
        compiler_params=pltpu.CompilerParams(use_tc_tiling_on_sc=False),
    )(src, dst, g)


def _sc_scatter_body(src_hbm, dst_hbm, g_hbm, acc_out, si_v, di_v, rows_v,
                     buf_v, acc_sh, sem):
    c = lax.axis_index("c")
    s = lax.axis_index("s")
    w = s * NC + c
    base = w * EPT
    for i in range(RPT):
        buf_v[i, :] = jnp.zeros((16,), jnp.float32)
    pltpu.sync_copy(buf_v, acc_sh.at[pl.ds(s * RPT, RPT)])
    for j in range(NCH):
        pltpu.sync_copy(src_hbm.at[pl.ds(base + j * 128, 128)], si_v.at[j])
        pltpu.sync_copy(dst_hbm.at[pl.ds(base + j * 128, 128)], di_v.at[j])
    waits = []
    for j in range(NCH):
        waits.append(
            pltpu.async_copy(
                g_hbm.at[si_v.at[j]], rows_v.at[pl.ds(j * 128, 128)], sem
            )
        )
    for wdma in waits:
        wdma.wait()
    plsc.subcore_barrier()
    for j in range(NCH):
        pltpu.sync_copy(
            rows_v.at[pl.ds(j * 128, 128)], acc_sh.at[di_v.at[j]], add=True
        )
    plsc.subcore_barrier()
    pltpu.sync_copy(
        acc_sh.at[pl.ds(s * RPT, RPT)], acc_out.at[c, pl.ds(s * RPT, RPT)]
    )


def _ab_body(acc_ref, g_ref, dinv_ref, b_ref, w1a_ref, w1b_ref, b1_ref,
             a_ref, bb_ref):
    pre = dinv_ref[...] * (acc_ref[0] + acc_ref[1] + g_ref[...]) + b_ref[...]
    h = jnp.maximum(pre, 0.0)
    a_ref[...] = _dot(h, w1a_ref[...]) + b1_ref[...]
    bb_ref[...] = _dot(h, w1b_ref[...])


def _ab(acc_p, g, dinv, b, W1, b1):
    return pl.pallas_call(
        _ab_body,
        out_shape=(
            jax.ShapeDtypeStruct((N, F), jnp.float32),
            jax.ShapeDtypeStruct((N, F), jnp.float32),
        ),
    )(acc_p, g, dinv, b.reshape(1, F), W1[:F], W1[F:], b1.reshape(1, F))


def _tile_score(x_rows, yt_cols, w2r):
    acc = jnp.zeros((BM, BN), jnp.float32)
    for k in range(F):
        t = x_rows[:, k : k + 1] + yt_cols[k : k + 1, :]
        tr = jnp.maximum(t, 0.0).astype(jnp.bfloat16).astype(jnp.float32)
        acc = acc + w2r[0, k] * tr
    return acc


def _pairs_body(a_i, bt_j, b_i, at_j, w2, b2, out_ref):
    ib = pl.program_id(0)
    jb = pl.program_id(1)
    w2r = w2[...]
    thr = -b2[0, 0]

    def upper():
        return (_tile_score(a_i[...], bt_j[...], w2r) > thr).astype(jnp.float32)

    def lower():
        return (_tile_score(b_i[...], at_j[...], w2r) > thr).astype(jnp.float32)

    def diag():
        rows = ib * BM + lax.broadcasted_iota(jnp.int32, (BM, BN), 0)
        cols = jb * BN + lax.broadcasted_iota(jnp.int32, (BM, BN), 1)
        acc = jnp.where(rows < cols,
                        _tile_score(a_i[...], bt_j[...], w2r),
                        _tile_score(b_i[...], at_j[...], w2r))
        hit = jnp.logical_and(acc > thr, rows != cols)
        return hit.astype(jnp.float32)

    row0 = ib * BM
    col0 = jb * BN
    out_ref[...] = lax.cond(
        col0 >= row0 + BM, upper,
        lambda: lax.cond(col0 + BN <= row0, lower, diag),
    )


def _all_pairs(A, B, W2, b2):
    w2 = _rne_bf16(W2.reshape(1, F))
    b2 = b2.reshape(1, 1)
    grid = (N // BM, N // BN)
    return pl.pallas_call(
        _pairs_body,
        grid=grid,
        in_specs=[
            pl.BlockSpec((BM, F), lambda i, j: (i, 0)),
            pl.BlockSpec((F, BN), lambda i, j: (0, j)),
            pl.BlockSpec((BM, F), lambda i, j: (i, 0)),
            pl.BlockSpec((F, BN), lambda i, j: (0, j)),
            pl.BlockSpec((1, F), lambda i, j: (0, 0)),
            pl.BlockSpec((1, 1), lambda i, j: (0, 0)),
        ],
        out_specs=pl.BlockSpec((BM, BN), lambda i, j: (i, j)),
        out_shape=jax.ShapeDtypeStruct((N, N), jnp.float32),
    )(A, B.T, B, A.T, w2, b2)


def kernel(x, edge_index, W, b, W1, b1, W2, b2):
    src = edge_index[0]
    dst = edge_index[1]
    deg_p = _sc_hist(dst)
    g, dinv = _prep(deg_p, x, W)
    acc_p = _sc_scatter(src, dst, g)
    A, B = _ab(acc_p, g, dinv, b, W1, b1)
    return _all_pairs(A, B, W2, b2)


# speedup vs baseline: 192.9878x; 1.2680x over previous
import functools

import jax
import jax.numpy as jnp
from jax import lax
from jax.experimental import pallas as pl
from jax.experimental.pallas import tpu as pltpu
from jax.experimental.pallas import tpu_sc as plsc

N = 2048
E = 65536
D = 128
F = 16
BM = 128
BN = 128

NC = 2
NS = 16
NW = NC * NS
EPT = E // NW
NCH = EPT // 128
RPT = N // NS

@functools.cache
def _mesh():
    return plsc.VectorSubcoreMesh(core_axis_name="c", subcore_axis_name="s")


def _rne_bf16(v):
    u = lax.bitcast_convert_type(v, jnp.uint32)
    u = u + jnp.uint32(0x7FFF) + ((u >> 16) & jnp.uint32(1))
    u = u & jnp.uint32(0xFFFF0000)
    return lax.bitcast_convert_type(u, jnp.float32)


def _dot(a, b):
    return lax.dot_general(
        _rne_bf16(a), _rne_bf16(b), (((1,), (0,)), ((), ())),
        precision=lax.Precision.HIGHEST,
    )


def _sc_hist(dst):
    return pl.kernel(
        _sc_hist_body,
        out_type=jax.ShapeDtypeStruct((NC, N), jnp.float32),
        mesh=_mesh(),
        scratch_types=[
            pltpu.VMEM((NCH, 128), jnp.int32),
            pltpu.VMEM((128,), jnp.float32),
            pltpu.VMEM_SHARED((N,), jnp.float32),
        ],
    )(dst)


def _sc_hist_body(dst_hbm, deg_out, idx_v, ones_v, deg_sh):
    c = lax.axis_index("c")
    s = lax.axis_index("s")
    w = s * NC + c
    base = w * EPT
    for i in range(8):
        ones_v[pl.ds(i * 16, 16)] = jnp.zeros((16,), jnp.float32)
    pltpu.sync_copy(ones_v, deg_sh.at[pl.ds(s * RPT, RPT)])
    for i in range(8):
        ones_v[pl.ds(i * 16, 16)] = jnp.ones((16,), jnp.float32)
    for j in range(NCH):
        pltpu.sync_copy(dst_hbm.at[pl.ds(base + j * 128, 128)], idx_v.at[j])
    plsc.subcore_barrier()
    for j in range(NCH):
        pltpu.sync_copy(ones_v, deg_sh.at[idx_v.at[j]], add=True)
    plsc.subcore_barrier()
    pltpu.sync_copy(
        deg_sh.at[pl.ds(s * RPT, RPT)], deg_out.at[c, pl.ds(s * RPT, RPT)]
    )


def _prep_body(deg_ref, x_ref, w_ref, g_ref, dinv_ref):
    deg = deg_ref[0:1, :] + deg_ref[1:2, :] + 1.0
    dinv = jnp.reshape(lax.rsqrt(deg), (N, 1))
    g_ref[...] = dinv * _dot(x_ref[...], w_ref[...])
    dinv_ref[...] = dinv


def _prep(deg_p, x, W):
    return pl.pallas_call(
        _prep_body,
        out_shape=(
            jax.ShapeDtypeStruct((N, F), jnp.float32),
            jax.ShapeDtypeStruct((N, 1), jnp.float32),
        ),
    )(deg_p, x, W)


def _sc_scatter(src, dst, g):
    return pl.kernel(
        _sc_scatter_body,
        out_type=jax.ShapeDtypeStruct((NC, N, F), jnp.float32),
        mesh=_mesh(),
        scratch_types=[
            pltpu.VMEM((NCH, 128), jnp.int32),
            pltpu.VMEM((NCH, 128), jnp.int32),
            pltpu.VMEM((EPT, F), jnp.float32),
            pltpu.VMEM((RPT, F), jnp.float32),
            pltpu.VMEM_SHARED((N, F), jnp.float32),
            pltpu.SemaphoreType.DMA,
        ],
        compiler_params=pltpu.CompilerParams(use_tc_tiling_on_sc=False),
    )(src, dst, g)


def _sc_scatter_body(src_hbm, dst_hbm, g_hbm, acc_out, si_v, di_v, rows_v,
                     buf_v, acc_sh, sem):
    c = lax.axis_index("c")
    s = lax.axis_index("s")
    w = s * NC + c
    base = w * EPT
    for i in range(RPT):
        buf_v[i, :] = jnp.zeros((16,), jnp.float32)
    pltpu.sync_copy(buf_v, acc_sh.at[pl.ds(s * RPT, RPT)])
    for j in range(NCH):
        pltpu.sync_copy(src_hbm.at[pl.ds(base + j * 128, 128)], si_v.at[j])
        pltpu.sync_copy(dst_hbm.at[pl.ds(base + j * 128, 128)], di_v.at[j])
    waits = []
    for j in range(NCH):
        waits.append(
            pltpu.async_copy(
                g_hbm.at[si_v.at[j]], rows_v.at[pl.ds(j * 128, 128)], sem
            )
        )
    for wdma in waits:
        wdma.wait()
    plsc.subcore_barrier()
    for j in range(NCH):
        pltpu.sync_copy(
            rows_v.at[pl.ds(j * 128, 128)], acc_sh.at[di_v.at[j]], add=True
        )
    plsc.subcore_barrier()
    pltpu.sync_copy(
        acc_sh.at[pl.ds(s * RPT, RPT)], acc_out.at[c, pl.ds(s * RPT, RPT)]
    )


def _ab_body(acc_ref, g_ref, dinv_ref, b_ref, w1a_ref, w1b_ref, b1_ref,
             a_ref, bb_ref):
    pre = dinv_ref[...] * (acc_ref[0] + acc_ref[1] + g_ref[...]) + b_ref[...]
    h = jnp.maximum(pre, 0.0)
    a_ref[...] = _dot(h, w1a_ref[...]) + b1_ref[...]
    bb_ref[...] = _dot(h, w1b_ref[...])


def _ab(acc_p, g, dinv, b, W1, b1):
    return pl.pallas_call(
        _ab_body,
        out_shape=(
            jax.ShapeDtypeStruct((N, F), jnp.float32),
            jax.ShapeDtypeStruct((N, F), jnp.float32),
        ),
    )(acc_p, g, dinv, b.reshape(1, F), W1[:F], W1[F:], b1.reshape(1, F))


def _tile_score(x_rows, yt_cols, w2r):
    acc = jnp.zeros((BM, BN), jnp.float32)
    for k in range(F):
        t = x_rows[:, k : k + 1] + yt_cols[k : k + 1, :]
        tr = jnp.maximum(t, 0.0).astype(jnp.bfloat16).astype(jnp.float32)
        acc = acc + w2r[0, k] * tr
    return acc


def _pairs_body(a_i, b_i, bt, at, w2, b2, out_ref):
    ib = pl.program_id(0)
    w2r = w2[...]
    thr = -b2[0, 0]
    a_rows = a_i[...]
    b_rows = b_i[...]
    for c in range(N // BN):
        c0 = c * BN

        def upper(c0=c0):
            s = _tile_score(a_rows, bt[:, c0 : c0 + BN], w2r)
            return (s > thr).astype(jnp.float32)

        def lower(c0=c0):
            s = _tile_score(b_rows, at[:, c0 : c0 + BN], w2r)
            return (s > thr).astype(jnp.float32)

        def diag(c0=c0):
            rows = lax.broadcasted_iota(jnp.int32, (BM, BN), 0)
            cols = lax.broadcasted_iota(jnp.int32, (BM, BN), 1)
            s = jnp.where(rows < cols,
                          _tile_score(a_rows, bt[:, c0 : c0 + BN], w2r),
                          _tile_score(b_rows, at[:, c0 : c0 + BN], w2r))
            hit = jnp.logical_and(s > thr, rows != cols)
            return hit.astype(jnp.float32)

        out_ref[:, c0 : c0 + BN] = lax.cond(
            ib < c, upper, lambda: lax.cond(ib > c, lower, diag)
        )


def _all_pairs(A, B, W2, b2):
    w2 = _rne_bf16(W2.reshape(1, F))
    b2 = b2.reshape(1, 1)
    return pl.pallas_call(
        _pairs_body,
        grid=(N // BM,),
        in_specs=[
            pl.BlockSpec((BM, F), lambda i: (i, 0)),
            pl.BlockSpec((BM, F), lambda i: (i, 0)),
            pl.BlockSpec((F, N), lambda i: (0, 0)),
            pl.BlockSpec((F, N), lambda i: (0, 0)),
            pl.BlockSpec((1, F), lambda i: (0, 0)),
            pl.BlockSpec((1, 1), lambda i: (0, 0)),
        ],
        out_specs=pl.BlockSpec((BM, N), lambda i: (i, 0)),
        out_shape=jax.ShapeDtypeStruct((N, N), jnp.float32),
    )(A, B, B.T, A.T, w2, b2)


def kernel(x, edge_index, W, b, W1, b1, W2, b2):
    src = edge_index[0]
    dst = edge_index[1]
    deg_p = _sc_hist(dst)
    g, dinv = _prep(deg_p, x, W)
    acc_p = _sc_scatter(src, dst, g)
    A, B = _ab(acc_p, g, dinv, b, W1, b1)
    return _all_pairs(A, B, W2, b2)


# speedup vs baseline: 216.5725x; 1.1222x over previous
import functools

import jax
import jax.numpy as jnp
from jax import lax
from jax.experimental import pallas as pl
from jax.experimental.pallas import tpu as pltpu
from jax.experimental.pallas import tpu_sc as plsc

N = 2048
E = 65536
D = 128
F = 16
BM = 128
BN = 128

NC = 2
NS = 16
NW = NC * NS
EPT = E // NW
NCH = EPT // 128
RPT = N // NS

@functools.cache
def _mesh():
    return plsc.VectorSubcoreMesh(core_axis_name="c", subcore_axis_name="s")


def _rne_bf16(v):
    u = lax.bitcast_convert_type(v, jnp.uint32)
    u = u + jnp.uint32(0x7FFF) + ((u >> 16) & jnp.uint32(1))
    u = u & jnp.uint32(0xFFFF0000)
    return lax.bitcast_convert_type(u, jnp.float32)


def _dot(a, b):
    return lax.dot_general(
        _rne_bf16(a), _rne_bf16(b), (((1,), (0,)), ((), ())),
        precision=lax.Precision.HIGHEST,
    )


def _sc_hist(dst):
    return pl.kernel(
        _sc_hist_body,
        out_type=jax.ShapeDtypeStruct((NC, N), jnp.float32),
        mesh=_mesh(),
        scratch_types=[
            pltpu.VMEM((NCH, 128), jnp.int32),
            pltpu.VMEM((128,), jnp.float32),
            pltpu.VMEM_SHARED((N,), jnp.float32),
            pltpu.SemaphoreType.DMA,
        ],
    )(dst.reshape(E // 128, 128))


def _sc_hist_body(dst_hbm, deg_out, idx_v, ones_v, deg_sh, sem):
    c = lax.axis_index("c")
    s = lax.axis_index("s")
    w = s * NC + c
    for i in range(8):
        ones_v[pl.ds(i * 16, 16)] = jnp.zeros((16,), jnp.float32)
    pltpu.sync_copy(ones_v, deg_sh.at[pl.ds(s * RPT, RPT)])
    for i in range(8):
        ones_v[pl.ds(i * 16, 16)] = jnp.ones((16,), jnp.float32)
    pltpu.sync_copy(dst_hbm.at[pl.ds(w * NCH, NCH)], idx_v)
    plsc.subcore_barrier()
    waits = [
        pltpu.async_copy(ones_v, deg_sh.at[idx_v.at[j]], sem, add=True)
        for j in range(NCH)
    ]
    for wdma in waits:
        wdma.wait()
    plsc.subcore_barrier()
    pltpu.sync_copy(
        deg_sh.at[pl.ds(s * RPT, RPT)], deg_out.at[c, pl.ds(s * RPT, RPT)]
    )


def _prep_body(deg_ref, x_ref, w_ref, g_ref, dinv_ref):
    deg = deg_ref[0:1, :] + deg_ref[1:2, :] + 1.0
    dinv = jnp.reshape(lax.rsqrt(deg), (N, 1))
    g_ref[...] = dinv * _dot(x_ref[...], w_ref[...])
    dinv_ref[...] = dinv


def _prep(deg_p, x, W):
    return pl.pallas_call(
        _prep_body,
        out_shape=(
            jax.ShapeDtypeStruct((N, F), jnp.float32),
            jax.ShapeDtypeStruct((N, 1), jnp.float32),
        ),
    )(deg_p, x, W)


def _sc_scatter(src, dst, g):
    return pl.kernel(
        _sc_scatter_body,
        out_type=jax.ShapeDtypeStruct((NC, N, F), jnp.float32),
        mesh=_mesh(),
        scratch_types=[
            pltpu.VMEM((NCH, 128), jnp.int32),
            pltpu.VMEM((NCH, 128), jnp.int32),
            pltpu.VMEM((EPT, F), jnp.float32),
            pltpu.VMEM((RPT, F), jnp.float32),
            pltpu.VMEM_SHARED((N, F), jnp.float32),
            pltpu.SemaphoreType.DMA,
            pltpu.SemaphoreType.DMA,
        ],
        compiler_params=pltpu.CompilerParams(use_tc_tiling_on_sc=False),
    )(src.reshape(E // 128, 128), dst.reshape(E // 128, 128), g)


def _sc_scatter_body(src_hbm, dst_hbm, g_hbm, acc_out, si_v, di_v, rows_v,
                     buf_v, acc_sh, sem, sem2):
    c = lax.axis_index("c")
    s = lax.axis_index("s")
    w = s * NC + c
    for i in range(RPT):
        buf_v[i, :] = jnp.zeros((16,), jnp.float32)
    pltpu.sync_copy(buf_v, acc_sh.at[pl.ds(s * RPT, RPT)])
    pltpu.sync_copy(src_hbm.at[pl.ds(w * NCH, NCH)], si_v)
    pltpu.sync_copy(dst_hbm.at[pl.ds(w * NCH, NCH)], di_v)
    waits = []
    for j in range(NCH):
        waits.append(
            pltpu.async_copy(
                g_hbm.at[si_v.at[j]], rows_v.at[pl.ds(j * 128, 128)], sem
            )
        )
    for wdma in waits:
        wdma.wait()
    plsc.subcore_barrier()
    waits = []
    for j in range(NCH):
        waits.append(
            pltpu.async_copy(
                rows_v.at[pl.ds(j * 128, 128)], acc_sh.at[di_v.at[j]], sem2,
                add=True,
            )
        )
    for wdma in waits:
        wdma.wait()
    plsc.subcore_barrier()
    pltpu.sync_copy(
        acc_sh.at[pl.ds(s * RPT, RPT)], acc_out.at[c, pl.ds(s * RPT, RPT)]
    )


def _ab_body(acc_ref, g_ref, dinv_ref, b_ref, w1a_ref, w1b_ref, b1_ref,
             a_ref, bb_ref):
    pre = dinv_ref[...] * (acc_ref[0] + acc_ref[1] + g_ref[...]) + b_ref[...]
    h = jnp.maximum(pre, 0.0)
    a_ref[...] = _dot(h, w1a_ref[...]) + b1_ref[...]
    bb_ref[...] = _dot(h, w1b_ref[...])


def _ab(acc_p, g, dinv, b, W1, b1):
    return pl.pallas_call(
        _ab_body,
        out_shape=(
            jax.ShapeDtypeStruct((N, F), jnp.float32),
            jax.ShapeDtypeStruct((N, F), jnp.float32),
        ),
    )(acc_p, g, dinv, b.reshape(1, F), W1[:F], W1[F:], b1.reshape(1, F))


def _tile_score(x_rows, yt_cols, w2r):
    acc = jnp.zeros((BM, BN), jnp.float32)
    for k in range(F):
        t = x_rows[:, k : k + 1] + yt_cols[k : k + 1, :]
        r = jnp.maximum(t, 0.0)
        if k % 2 == 0:
            tr = r.astype(jnp.bfloat16).astype(jnp.float32)
        else:
            tr = _rne_bf16(r)
        acc = acc + w2r[0, k] * tr
    return acc


def _pairs_body(a_i, b_i, bt, at, w2, b2, out_ref):
    ib = pl.program_id(0)
    w2r = w2[...]
    thr = -b2[0, 0]
    a_rows = a_i[...]
    b_rows = b_i[...]
    for c in range(N // BN):
        c0 = c * BN

        def upper(c0=c0):
            s = _tile_score(a_rows, bt[:, c0 : c0 + BN], w2r)
            return (s > thr).astype(jnp.float32)

        def lower(c0=c0):
            s = _tile_score(b_rows, at[:, c0 : c0 + BN], w2r)
            return (s > thr).astype(jnp.float32)

        def diag(c0=c0):
            rows = lax.broadcasted_iota(jnp.int32, (BM, BN), 0)
            cols = lax.broadcasted_iota(jnp.int32, (BM, BN), 1)
            s = jnp.where(rows < cols,
                          _tile_score(a_rows, bt[:, c0 : c0 + BN], w2r),
                          _tile_score(b_rows, at[:, c0 : c0 + BN], w2r))
            hit = jnp.logical_and(s > thr, rows != cols)
            return hit.astype(jnp.float32)

        out_ref[:, c0 : c0 + BN] = lax.cond(
            ib < c, upper, lambda: lax.cond(ib > c, lower, diag)
        )


def _all_pairs(A, B, W2, b2):
    w2 = _rne_bf16(W2.reshape(1, F))
    b2 = b2.reshape(1, 1)
    return pl.pallas_call(
        _pairs_body,
        grid=(N // BM,),
        in_specs=[
            pl.BlockSpec((BM, F), lambda i: (i, 0)),
            pl.BlockSpec((BM, F), lambda i: (i, 0)),
            pl.BlockSpec((F, N), lambda i: (0, 0)),
            pl.BlockSpec((F, N), lambda i: (0, 0)),
            pl.BlockSpec((1, F), lambda i: (0, 0)),
            pl.BlockSpec((1, 1), lambda i: (0, 0)),
        ],
        out_specs=pl.BlockSpec((BM, N), lambda i: (i, 0)),
        out_shape=jax.ShapeDtypeStruct((N, N), jnp.float32),
    )(A, B, B.T, A.T, w2, b2)


def kernel(x, edge_index, W, b, W1, b1, W2, b2):
    src = edge_index[0]
    dst = edge_index[1]
    deg_p = _sc_hist(dst)
    g, dinv = _prep(deg_p, x, W)
    acc_p = _sc_scatter(src, dst, g)
    A, B = _ab(acc_p, g, dinv, b, W1, b1)
    return _all_pairs(A, B, W2, b2)
